# baseline (speedup 1.0000x reference)
import functools

import jax
import jax.numpy as jnp
from jax.experimental import pallas as pl
from jax.experimental.pallas import tpu as pltpu

R_CUT = 1.4415
N_RBF = 20

E_BLK = 4000


def _silu(a):
    return a * jax.nn.sigmoid(a)


def _msg1_body(xj_ref, ea2_ref, Wm1_ref, bm1_ref, Wm3_ref, bm3_ref,
               Wm2_ref, bm2_ref, out_ref):
    xj = xj_ref[...]
    s = xj[:, 48:]
    h = _silu(s @ Wm1_ref[...] + bm1_ref[...])
    s2 = h @ Wm3_ref[...] + bm3_ref[...]
    x = ea2_ref[...]
    xs = jnp.maximum(x, 1e-9)
    ns = 1.0 + jax.lax.broadcasted_iota(jnp.int32, (1, N_RBF), 1).astype(jnp.float32)
    basis = jnp.sqrt(2.0 / R_CUT) * jnp.sin(ns * (jnp.pi / R_CUT) * xs) / xs
    cut = 0.5 * (jnp.cos(jnp.pi * x / R_CUT) + 1.0) * (x < R_CUT).astype(jnp.float32)
    eb = (basis * cut) @ Wm2_ref[...] + bm2_ref[...]
    split = s2 * eb
    sp1 = split[:, :16]
    out_ref[:, 0:16] = xj[:, 0:16] * sp1
    out_ref[:, 16:32] = xj[:, 16:32] * sp1
    out_ref[:, 32:48] = xj[:, 32:48] * sp1
    out_ref[:, 48:64] = split[:, 16:32]


def _msg2_body(xj_ref, U_ref, V_ref, Wu1_ref, bu1_ref, Wu2_ref, bu2_ref,
               out_ref):
    xj = xj_ref[...]
    Um = U_ref[...]
    Vm = V_ref[...]
    v0 = xj[:, 0:16] @ Um
    v1 = xj[:, 16:32] @ Um
    v2 = xj[:, 32:48] @ Um
    s = xj[:, 48:64]
    vV0 = v0 @ Vm
    vV1 = v1 @ Vm
    vV2 = v2 @ Vm
    norm = jnp.sqrt(vV0 * vV0 + vV1 * vV1 + vV2 * vV2)
    stack = jnp.concatenate([norm, s], axis=1)
    h = _silu(stack @ Wu1_ref[...] + bu1_ref[...])
    split = h @ Wu2_ref[...] + bu2_ref[...]
    u1 = split[:, 0:16]
    u3 = split[:, 32:48]
    w0 = u1 * v0
    w1 = u1 * v1
    w2 = u1 * v2
    t0 = w0 * vV0
    t1 = w1 * vV1
    t2 = w2 * vV2
    vV = jnp.sqrt(t0 * t0 + t1 * t1 + t2 * t2)
    out_ref[:, 0:16] = w0
    out_ref[:, 16:32] = w1
    out_ref[:, 32:48] = w2
    out_ref[:, 48:64] = u3 + vV


def _edge_grid_specs(n_w, e_blk):
    wspec = [pl.BlockSpec(w_shape, lambda i: (0,) * len(w_shape))
             for w_shape in n_w]
    return wspec


def _msg1(xj, ea2, Wm1, bm1, Wm3, bm3, Wm2, bm2):
    e = xj.shape[0]
    grid = (e // E_BLK,)
    blk = lambda shape: pl.BlockSpec(shape, lambda i: (0, 0))
    return pl.pallas_call(
        _msg1_body,
        grid=grid,
        in_specs=[
            pl.BlockSpec((E_BLK, 64), lambda i: (i, 0)),
            pl.BlockSpec((E_BLK, 1), lambda i: (i, 0)),
            blk((16, 16)), blk((1, 16)), blk((16, 48)), blk((1, 48)),
            blk((N_RBF, 48)), blk((1, 48)),
        ],
        out_specs=pl.BlockSpec((E_BLK, 64), lambda i: (i, 0)),
        out_shape=jax.ShapeDtypeStruct((e, 64), jnp.float32),
    )(xj, ea2, Wm1, bm1.reshape(1, 16), Wm3, bm3.reshape(1, 48),
      Wm2, bm2.reshape(1, 48))


def _msg2(xj, U, V, Wu1, bu1, Wu2, bu2):
    e = xj.shape[0]
    grid = (e // E_BLK,)
    blk = lambda shape: pl.BlockSpec(shape, lambda i: (0, 0))
    return pl.pallas_call(
        _msg2_body,
        grid=grid,
        in_specs=[
            pl.BlockSpec((E_BLK, 64), lambda i: (i, 0)),
            blk((16, 16)), blk((16, 16)),
            blk((32, 16)), blk((1, 16)), blk((16, 48)), blk((1, 48)),
        ],
        out_specs=pl.BlockSpec((E_BLK, 64), lambda i: (i, 0)),
        out_shape=jax.ShapeDtypeStruct((e, 64), jnp.float32),
    )(xj, U, V, Wu1, bu1.reshape(1, 16), Wu2, bu2.reshape(1, 48))


def kernel(x, edge_index, edge_attr1, edge_attr2, Wm1, bm1, Wm2, bm2,
           Wm3, bm3, U, V, Wu1, bu1, Wu2, bu2):
    src = edge_index[0]
    dst = edge_index[1]
    n = x.shape[0]
    ea2 = edge_attr2.reshape(-1, 1)

    xj = x[src]
    msg = _msg1(xj, ea2, Wm1, bm1, Wm3, bm3, Wm2, bm2)
    x1 = x + jax.ops.segment_sum(msg, dst, num_segments=n)

    xj2 = x1[src]
    msg2 = _msg2(xj2, U, V, Wu1, bu1, Wu2, bu2)
    out = x1 + jax.ops.segment_sum(msg2, dst, num_segments=n)
    return out


# speedup vs baseline: 1.4883x; 1.4883x over previous
import functools

import jax
import jax.numpy as jnp
from jax import lax
from jax.experimental import pallas as pl
from jax.experimental.pallas import tpu as pltpu
from jax.experimental.pallas import tpu_sc as plsc

R_CUT = 1.4415
N_RBF = 20

E_BLK = 4000

_NC = 2
_NS = 16
_NW = _NC * _NS
_CG = 128


def _sc_gather(x, src):
    e = src.shape[0]
    d = x.shape[1]
    nchunk = e // _CG
    trips = -(-nchunk // _NW)
    mesh = plsc.VectorSubcoreMesh(core_axis_name="c", subcore_axis_name="s")

    @functools.partial(
        pl.kernel,
        mesh=mesh,
        out_type=jax.ShapeDtypeStruct((e, d), jnp.float32),
        scratch_types=[
            pltpu.VMEM((_CG,), jnp.int32),
            pltpu.VMEM((_CG, d), jnp.float32),
            pltpu.SemaphoreType.DMA,
        ],
        compiler_params=pltpu.CompilerParams(use_tc_tiling_on_sc=False),
    )
    def k(x_hbm, src_hbm, out_hbm, idx_v, rows_v, sem):
        wid = lax.axis_index("s") * _NC + lax.axis_index("c")

        def body(i, carry):
            chunk = wid + i * _NW

            @pl.when(chunk < nchunk)
            def _():
                base = chunk * _CG
                pltpu.sync_copy(src_hbm.at[pl.ds(base, _CG)], idx_v)
                pltpu.async_copy(x_hbm.at[idx_v], rows_v, sem).wait()
                pltpu.sync_copy(rows_v, out_hbm.at[pl.ds(base, _CG)])

            return carry

        lax.fori_loop(0, trips, body, 0)

    return k(x, src)


def _silu(a):
    return a * jax.nn.sigmoid(a)


_CS = 128


def _sc_scatter_add(base_nodes, msg, dst):
    n, d = base_nodes.shape
    e = dst.shape[0]
    nchunk = e // _CS
    trips = -(-nchunk // _NS)
    half = d // 2
    rows_per_tile = n // _NS
    mesh = plsc.VectorSubcoreMesh(core_axis_name="c", subcore_axis_name="s")

    @functools.partial(
        pl.kernel,
        mesh=mesh,
        out_type=jax.ShapeDtypeStruct((n, d), jnp.float32),
        scratch_types=[
            pltpu.VMEM((_CS,), jnp.int32),
            pltpu.VMEM((_CS, 32), jnp.float32),
            pltpu.VMEM_SHARED((n, 32), jnp.float32),
            pltpu.SemaphoreType.DMA,
        ],
        compiler_params=pltpu.CompilerParams(use_tc_tiling_on_sc=False),
    )
    def k(xb_hbm, msg_hbm, dst_hbm, out_hbm, idx_v, upd_v, acc_sh, sem):
        c = lax.axis_index("c")
        s = lax.axis_index("s")
        r0 = s * rows_per_tile
        pltpu.sync_copy(
            xb_hbm.at[pl.ds(r0, rows_per_tile), pl.ds(c * half, half)],
            acc_sh.at[pl.ds(r0, rows_per_tile)])
        plsc.subcore_barrier()

        def body(i, carry):
            chunk = s + i * _NS

            @pl.when(chunk < nchunk)
            def _():
                b = chunk * _CS
                pltpu.sync_copy(dst_hbm.at[pl.ds(b, _CS)], idx_v)
                pltpu.sync_copy(
                    msg_hbm.at[pl.ds(b, _CS), pl.ds(c * half, half)], upd_v)
                pltpu.sync_copy(upd_v, acc_sh.at[idx_v], add=True)

            return carry

        lax.fori_loop(0, trips, body, 0)
        plsc.subcore_barrier()
        pltpu.sync_copy(
            acc_sh.at[pl.ds(r0, rows_per_tile)],
            out_hbm.at[pl.ds(r0, rows_per_tile), pl.ds(c * half, half)])

    return k(base_nodes, msg, dst)


def _msg1_body(xj_ref, ea2_ref, Wm1_ref, bm1_ref, Wm3_ref, bm3_ref,
               Wm2_ref, bm2_ref, out_ref):
    xj = xj_ref[...]
    s = xj[:, 48:]
    h = _silu(s @ Wm1_ref[...] + bm1_ref[...])
    s2 = h @ Wm3_ref[...] + bm3_ref[...]
    x = ea2_ref[...]
    xs = jnp.maximum(x, 1e-9)
    ns = 1.0 + jax.lax.broadcasted_iota(jnp.int32, (1, N_RBF), 1).astype(jnp.float32)
    basis = jnp.sqrt(2.0 / R_CUT) * jnp.sin(ns * (jnp.pi / R_CUT) * xs) / xs
    cut = 0.5 * (jnp.cos(jnp.pi * x / R_CUT) + 1.0) * (x < R_CUT).astype(jnp.float32)
    eb = (basis * cut) @ Wm2_ref[...] + bm2_ref[...]
    split = s2 * eb
    sp1 = split[:, :16]
    out_ref[:, 0:16] = xj[:, 0:16] * sp1
    out_ref[:, 16:32] = xj[:, 16:32] * sp1
    out_ref[:, 32:48] = xj[:, 32:48] * sp1
    out_ref[:, 48:64] = split[:, 16:32]


def _msg2_body(xj_ref, U_ref, V_ref, Wu1_ref, bu1_ref, Wu2_ref, bu2_ref,
               out_ref):
    xj = xj_ref[...]
    Um = U_ref[...]
    Vm = V_ref[...]
    v0 = xj[:, 0:16] @ Um
    v1 = xj[:, 16:32] @ Um
    v2 = xj[:, 32:48] @ Um
    s = xj[:, 48:64]
    vV0 = v0 @ Vm
    vV1 = v1 @ Vm
    vV2 = v2 @ Vm
    norm = jnp.sqrt(vV0 * vV0 + vV1 * vV1 + vV2 * vV2)
    stack = jnp.concatenate([norm, s], axis=1)
    h = _silu(stack @ Wu1_ref[...] + bu1_ref[...])
    split = h @ Wu2_ref[...] + bu2_ref[...]
    u1 = split[:, 0:16]
    u3 = split[:, 32:48]
    w0 = u1 * v0
    w1 = u1 * v1
    w2 = u1 * v2
    t0 = w0 * vV0
    t1 = w1 * vV1
    t2 = w2 * vV2
    vV = jnp.sqrt(t0 * t0 + t1 * t1 + t2 * t2)
    out_ref[:, 0:16] = w0
    out_ref[:, 16:32] = w1
    out_ref[:, 32:48] = w2
    out_ref[:, 48:64] = u3 + vV


def _edge_grid_specs(n_w, e_blk):
    wspec = [pl.BlockSpec(w_shape, lambda i: (0,) * len(w_shape))
             for w_shape in n_w]
    return wspec


def _msg1(xj, ea2, Wm1, bm1, Wm3, bm3, Wm2, bm2):
    e = xj.shape[0]
    grid = (e // E_BLK,)
    blk = lambda shape: pl.BlockSpec(shape, lambda i: (0, 0))
    return pl.pallas_call(
        _msg1_body,
        grid=grid,
        in_specs=[
            pl.BlockSpec((E_BLK, 64), lambda i: (i, 0)),
            pl.BlockSpec((E_BLK, 1), lambda i: (i, 0)),
            blk((16, 16)), blk((1, 16)), blk((16, 48)), blk((1, 48)),
            blk((N_RBF, 48)), blk((1, 48)),
        ],
        out_specs=pl.BlockSpec((E_BLK, 64), lambda i: (i, 0)),
        out_shape=jax.ShapeDtypeStruct((e, 64), jnp.float32),
    )(xj, ea2, Wm1, bm1.reshape(1, 16), Wm3, bm3.reshape(1, 48),
      Wm2, bm2.reshape(1, 48))


def _msg2(xj, U, V, Wu1, bu1, Wu2, bu2):
    e = xj.shape[0]
    grid = (e // E_BLK,)
    blk = lambda shape: pl.BlockSpec(shape, lambda i: (0, 0))
    return pl.pallas_call(
        _msg2_body,
        grid=grid,
        in_specs=[
            pl.BlockSpec((E_BLK, 64), lambda i: (i, 0)),
            blk((16, 16)), blk((16, 16)),
            blk((32, 16)), blk((1, 16)), blk((16, 48)), blk((1, 48)),
        ],
        out_specs=pl.BlockSpec((E_BLK, 64), lambda i: (i, 0)),
        out_shape=jax.ShapeDtypeStruct((e, 64), jnp.float32),
    )(xj, U, V, Wu1, bu1.reshape(1, 16), Wu2, bu2.reshape(1, 48))


def kernel(x, edge_index, edge_attr1, edge_attr2, Wm1, bm1, Wm2, bm2,
           Wm3, bm3, U, V, Wu1, bu1, Wu2, bu2):
    src = edge_index[0]
    dst = edge_index[1]
    n = x.shape[0]
    ea2 = edge_attr2.reshape(-1, 1)

    xj = _sc_gather(x, src)
    msg = _msg1(xj, ea2, Wm1, bm1, Wm3, bm3, Wm2, bm2)
    x1 = _sc_scatter_add(x, msg, dst)

    xj2 = _sc_gather(x1, src)
    msg2 = _msg2(xj2, U, V, Wu1, bu1, Wu2, bu2)
    out = _sc_scatter_add(x1, msg2, dst)
    return out
